# baseline (speedup 1.0000x reference)
import functools
import jax
import jax.numpy as jnp
import numpy as np
from jax import lax
from jax.experimental import pallas as pl
from jax.experimental.pallas import tpu as pltpu
from jax.experimental.pallas import tpu_sc as plsc

_NUM_EMBED = 384
_N = 200000
_B = 128
_R = 3 * _B
_NBLK_FULL = _N // _B
_NC, _NS, _L = 2, 16, 16
_NW = _NC * _NS
_T = 49
_NBLK = _NW * _T


def _body(fused_hbm, cidx_hbm, out_hbm, shared,
          idx0, idx1, buf0, buf1,
          sem_i0, sem_i1, sem_g0, sem_g1, sem_w0, sem_w1):
  cid = lax.axis_index("c")
  sid = lax.axis_index("s")
  wid = sid * _NC + cid

  idxv = (idx0, idx1)
  buf = (buf0, buf1)
  sem_i = (sem_i0, sem_i1)
  sem_g = (sem_g0, sem_g1)
  sem_w = (sem_w0, sem_w1)

  @pl.when(sid == 0)
  def _stage():
    pltpu.sync_copy(fused_hbm, shared)

  plsc.subcore_barrier()

  def idx_desc(t, s):
    b = wid + t * _NW
    return pltpu.make_async_copy(cidx_hbm.at[pl.ds(_R * b, _R)], idxv[s],
                                 sem_i[s])

  def start_idx(t, s):
    b = wid + t * _NW

    @pl.when(b < _NBLK)
    def _():
      idx_desc(t, s).start()

  def gather_descs(s):
    return [
        pltpu.make_async_copy(shared.at[idxv[s].at[pl.ds(_B * j, _B)]],
                              buf[s].at[pl.ds(_B * j, _B)], sem_g[s])
        for j in range(3)
    ]

  def write_desc(t, s):
    b = wid + t * _NW
    nb = jnp.minimum(b, _NBLK_FULL) * _B
    nb = nb - jnp.where(b == _NBLK_FULL, _B // 2, 0)
    nb = jnp.where(b > _NBLK_FULL, 0, nb)
    return pltpu.make_async_copy(buf[s], out_hbm.at[pl.ds(3 * nb, _R)],
                                 sem_w[s])

  def step(t, s, first=False):
    if not first:
      write_desc(t - 2, s).wait()
    idx_desc(t, s).wait()
    for c in gather_descs(s):
      c.start()
    for c in gather_descs(s):
      c.wait()
    write_desc(t, s).start()
    start_idx(t + 2, s)

  start_idx(0, 0)
  start_idx(1, 1)
  step(0, 0, first=True)
  step(1, 1, first=True)

  def loop_body(i, carry):
    step(2 * i, 0)
    step(2 * i + 1, 1)
    return carry

  lax.fori_loop(1, (_T - 1) // 2, loop_body, 0)

  step(_T - 1, 0)
  write_desc(_T - 2, 1).wait()
  write_desc(_T - 1, 0).wait()


@jax.jit
def _run(fused, cidx):
  mesh = plsc.VectorSubcoreMesh(core_axis_name="c", subcore_axis_name="s")
  return pl.kernel(
      _body,
      out_type=jax.ShapeDtypeStruct((3 * _N, 128), jnp.float32),
      mesh=mesh,
      scratch_types=[
          pltpu.VMEM_SHARED((3 * 512, 128), jnp.float32),
          pltpu.VMEM((_R,), jnp.int32),
          pltpu.VMEM((_R,), jnp.int32),
          pltpu.VMEM((_R, 128), jnp.float32),
          pltpu.VMEM((_R, 128), jnp.float32),
          pltpu.SemaphoreType.DMA,
          pltpu.SemaphoreType.DMA,
          pltpu.SemaphoreType.DMA,
          pltpu.SemaphoreType.DMA,
          pltpu.SemaphoreType.DMA,
          pltpu.SemaphoreType.DMA,
      ],
  )(fused, cidx)


def kernel(data, xyz, depth_idx, absolute_emb, depth_table):
  del data
  tabs = [absolute_emb[:, a::3] for a in range(3)]
  dchunks = [depth_table[:, 128 * a:128 * (a + 1)] for a in range(3)]
  fused = jnp.concatenate(
      [(t[:, None, :] + dc[None, :, :]).reshape(512, 128)
       for t, dc in zip(tabs, dchunks)], axis=0)

  off = jnp.tile(jnp.arange(3, dtype=jnp.int32) * 512, _N)
  cidx = xyz.reshape(-1) * 4 + jnp.repeat(depth_idx, 3) + off

  mainf = cidx[:_NBLK_FULL * _R]
  tailf = cidx[(_N - _B) * 3:]
  fillf = jnp.tile(cidx[:_R], (_NBLK - _NBLK_FULL - 1,))
  cidx_flat = jnp.concatenate([mainf, tailf, fillf])

  out = _run(fused, cidx_flat)
  return out.reshape(_N, _NUM_EMBED)


# speedup vs baseline: 4.3397x; 4.3397x over previous
import functools
import jax
import jax.numpy as jnp
import numpy as np
from jax import lax
from jax.experimental import pallas as pl
from jax.experimental.pallas import tpu as pltpu
from jax.experimental.pallas import tpu_sc as plsc

_NUM_EMBED = 384
_N = 200000
_B = 128
_NBLK_FULL = _N // _B
_NC, _NS, _L = 2, 16, 16
_NW = _NC * _NS
_T = 49
_NBLK = _NW * _T


def _body(fused_hbm, idx_hbm, out_hbm, shared, idxw,
          ix0, iy0, iz0, ix1, iy1, iz1,
          bx0, by0, bz0, bx1, by1, bz1,
          sem_i, sem_g0, sem_g1, sem_w0, sem_w1):
  cid = lax.axis_index("c")
  sid = lax.axis_index("s")
  wid = sid * _NC + cid

  ix = (ix0, ix1)
  iy = (iy0, iy1)
  iz = (iz0, iz1)
  bx = (bx0, bx1)
  by = (by0, by1)
  bz = (bz0, bz1)
  sem_g = (sem_g0, sem_g1)
  sem_w = (sem_w0, sem_w1)

  ci = pltpu.async_copy(idx_hbm.at[wid], idxw, sem_i)

  @pl.when(sid == 0)
  def _stage():
    pltpu.sync_copy(fused_hbm, shared)

  plsc.subcore_barrier()
  ci.wait()

  def out_base(t):
    b = wid + t * _NW
    base = jnp.minimum(b, _NBLK_FULL) * _B
    base = base - jnp.where(b == _NBLK_FULL, _B // 2, 0)
    return jnp.where(b > _NBLK_FULL, 0, base)

  def start_block(t, s):
    for g in range(_B // _L):
      sl = pl.ds(g * _L, _L)
      w = idxw[t, sl]
      d = lax.shift_right_logical(w, 24)
      px = w & 0xFF
      py = lax.shift_right_logical(w, 8) & 0xFF
      pz = lax.shift_right_logical(w, 16) & 0xFF
      ix[s][sl] = px * 4 + d
      iy[s][sl] = py * 4 + d + 512
      iz[s][sl] = pz * 4 + d + 1024
    pltpu.make_async_copy(shared.at[ix[s]], bx[s], sem_g[s]).start()
    pltpu.make_async_copy(shared.at[iy[s]], by[s], sem_g[s]).start()
    pltpu.make_async_copy(shared.at[iz[s]], bz[s], sem_g[s]).start()

  def wait_gathers(s):
    pltpu.make_async_copy(shared.at[ix[s]], bx[s], sem_g[s]).wait()
    pltpu.make_async_copy(shared.at[iy[s]], by[s], sem_g[s]).wait()
    pltpu.make_async_copy(shared.at[iz[s]], bz[s], sem_g[s]).wait()

  def write_descs(t, s):
    base = out_base(t)
    return (
        pltpu.make_async_copy(
            bx[s], out_hbm.at[pl.ds(base, _B), pl.ds(0, 128)], sem_w[s]),
        pltpu.make_async_copy(
            by[s], out_hbm.at[pl.ds(base, _B), pl.ds(128, 128)], sem_w[s]),
        pltpu.make_async_copy(
            bz[s], out_hbm.at[pl.ds(base, _B), pl.ds(256, 128)], sem_w[s]),
    )

  def issue_writes(t, s):
    for c in write_descs(t, s):
      c.start()

  def wait_writes(t, s):
    for c in write_descs(t, s):
      c.wait()

  start_block(0, 0)
  start_block(1, 1)
  wait_gathers(0)
  issue_writes(0, 0)
  wait_gathers(1)
  issue_writes(1, 1)

  def loop_body(i, carry):
    t0 = 2 * i
    wait_writes(t0 - 2, 0)
    start_block(t0, 0)
    wait_gathers(0)
    issue_writes(t0, 0)
    wait_writes(t0 - 1, 1)
    start_block(t0 + 1, 1)
    wait_gathers(1)
    issue_writes(t0 + 1, 1)
    return carry

  lax.fori_loop(1, (_T - 1) // 2, loop_body, 0)

  t_last = _T - 1
  wait_writes(t_last - 2, 0)
  start_block(t_last, 0)
  wait_gathers(0)
  issue_writes(t_last, 0)
  wait_writes(t_last - 1, 1)
  wait_writes(t_last, 0)


@jax.jit
def _run(fused, idx_packed):
  mesh = plsc.VectorSubcoreMesh(core_axis_name="c", subcore_axis_name="s")
  return pl.kernel(
      _body,
      out_type=jax.ShapeDtypeStruct((_N, _NUM_EMBED), jnp.float32),
      mesh=mesh,
      scratch_types=[
          pltpu.VMEM_SHARED((3 * 512, 128), jnp.float32),
          pltpu.VMEM((_T, _B), jnp.int32),
          pltpu.VMEM((_B,), jnp.int32),
          pltpu.VMEM((_B,), jnp.int32),
          pltpu.VMEM((_B,), jnp.int32),
          pltpu.VMEM((_B,), jnp.int32),
          pltpu.VMEM((_B,), jnp.int32),
          pltpu.VMEM((_B,), jnp.int32),
          pltpu.VMEM((_B, 128), jnp.float32),
          pltpu.VMEM((_B, 128), jnp.float32),
          pltpu.VMEM((_B, 128), jnp.float32),
          pltpu.VMEM((_B, 128), jnp.float32),
          pltpu.VMEM((_B, 128), jnp.float32),
          pltpu.VMEM((_B, 128), jnp.float32),
          pltpu.SemaphoreType.DMA,
          pltpu.SemaphoreType.DMA,
          pltpu.SemaphoreType.DMA,
          pltpu.SemaphoreType.DMA,
          pltpu.SemaphoreType.DMA,
      ],
  )(fused, idx_packed)


def kernel(data, xyz, depth_idx, absolute_emb, depth_table):
  del data
  tabs = [absolute_emb[:, a::3] for a in range(3)]
  dchunks = [depth_table[:, 128 * a:128 * (a + 1)] for a in range(3)]
  fused = jnp.concatenate(
      [(t[:, None, :] + dc[None, :, :]).reshape(512, 128)
       for t, dc in zip(tabs, dchunks)], axis=0)

  xyzT = xyz.T
  idxs = (xyzT[0] | (xyzT[1] << 8) | (xyzT[2] << 16)
          | (depth_idx << 24))
  main = idxs[:_NBLK_FULL * _B]
  tail = idxs[_N - _B:]
  n_fill = _NBLK - _NBLK_FULL - 1
  fill = jnp.tile(idxs[:_B], (n_fill,))
  blocks = jnp.concatenate([main, tail, fill], axis=0)
  blocks = blocks.reshape(_NBLK // _NW, _NW, _B)
  idx_packed = blocks.transpose(1, 0, 2)

  return _run(fused, idx_packed)
